# baseline (speedup 1.0000x reference)
import jax
import jax.numpy as jnp
from jax import lax
from jax.experimental import pallas as pl
from jax.experimental.pallas import tpu as pltpu
from jax.experimental.pallas import tpu_sc as plsc

B = 1024
V = 1000
VP = 1024
M = 32
H = 256
TIMING_DIM = 10
DECAY = 0.99

_NC, _NS = 2, 16
_NW = _NC * _NS
_ROWS = B * (M - 1)
_PER_W = _ROWS // _NW
_CHUNK = 32
_NCHUNK = _PER_W // _CHUNK


_BP = 64


def _pad_body(mem_ref, out_ref):
    out_ref[:, :, :V] = mem_ref[:, : M - 1, :]
    out_ref[:, :, V:] = jnp.zeros((_BP, M - 1, VP - V), jnp.float32)


def _pad_call(memory):
    return pl.pallas_call(
        _pad_body,
        grid=(B // _BP,),
        in_specs=[pl.BlockSpec((_BP, M, V), lambda i: (i, 0, 0))],
        out_specs=pl.BlockSpec((_BP, M - 1, VP), lambda i: (i, 0, 0)),
        out_shape=jax.ShapeDtypeStruct((B, M - 1, VP), jnp.float32),
        compiler_params=pltpu.CompilerParams(
            dimension_semantics=("arbitrary",)),
    )(memory)


def _sc_gather_body(idx_hbm, table_hbm, out_hbm, idx_v, rows_v, sem):
    wid = lax.axis_index("s") * _NC + lax.axis_index("c")
    for chunk in range(_NCHUNK):
        base = wid * _PER_W + chunk * _CHUNK
        pltpu.sync_copy(idx_hbm.at[pl.ds(base, _CHUNK)], idx_v)
        pltpu.async_copy(table_hbm.at[idx_v], rows_v, sem).wait()
        pltpu.sync_copy(rows_v, out_hbm.at[pl.ds(base, _CHUNK)])


def _sc_gather(idx, table):
    return pl.kernel(
        _sc_gather_body,
        out_type=jax.ShapeDtypeStruct((_ROWS, VP), jnp.float32),
        mesh=plsc.VectorSubcoreMesh(core_axis_name="c", subcore_axis_name="s"),
        scratch_types=[
            pltpu.VMEM((_CHUNK,), jnp.int32),
            pltpu.VMEM((_CHUNK, VP), jnp.float32),
            pltpu.SemaphoreType.DMA,
        ],
    )(idx, table)


def _mlp_body(x_ref, lp_ref, sm_ref, bt_ref, ss_ref,
              w1x_ref, w1b_ref, w1bt_ref, w1ss_ref, b1_ref,
              wgx_ref, wgb_ref, wgbt_ref, wgss_ref, bg_ref,
              w2_ref, b2_ref, out_ref):
    xb = x_ref[...].astype(jnp.bfloat16)
    a1 = jnp.dot(xb, w1x_ref[...], preferred_element_type=jnp.float32)
    a2 = jnp.dot(xb, wgx_ref[...], preferred_element_type=jnp.float32)
    for j in range(M - 1):
        smj = sm_ref[:, j, :].astype(jnp.bfloat16)
        a1 += jnp.dot(smj, w1b_ref[j], preferred_element_type=jnp.float32)
        a2 += jnp.dot(smj, wgb_ref[j], preferred_element_type=jnp.float32)
    btb = bt_ref[...].astype(jnp.bfloat16)
    a1 += jnp.dot(btb, w1bt_ref[...], preferred_element_type=jnp.float32)
    a2 += jnp.dot(btb, wgbt_ref[...], preferred_element_type=jnp.float32)
    surprise = -jnp.log(
        jnp.sum(x_ref[...] * lp_ref[...], axis=1, keepdims=True) + 1e-08)
    ss = ss_ref[...]
    a1 += jnp.dot(ss, w1ss_ref[...], preferred_element_type=jnp.float32)
    a2 += jnp.dot(ss, wgss_ref[...], preferred_element_type=jnp.float32)
    a1 += surprise * w1ss_ref[0:1, :]
    a2 += surprise * wgss_ref[0:1, :]
    h = (a1 + b1_ref[...]) * jax.nn.sigmoid(a2 + bg_ref[...])
    out_ref[...] = (
        jnp.dot(h.astype(jnp.bfloat16), w2_ref[...],
                preferred_element_type=jnp.float32) + b2_ref[...])


_BT = 64


def _full(shape):
    return pl.BlockSpec(shape, lambda i: (0,) * len(shape))


def _mlp_call(x, lp, sm, bt, ss, w1x, w1b, w1bt, w1ss, b1,
              wgx, wgb, wgbt, wgss, bg, w2, b2):
    return pl.pallas_call(
        _mlp_body,
        grid=(B // _BT,),
        in_specs=[
            pl.BlockSpec((_BT, V), lambda i: (i, 0)),
            pl.BlockSpec((_BT, V), lambda i: (i, 0)),
            pl.BlockSpec((_BT, M - 1, VP), lambda i: (i, 0, 0)),
            pl.BlockSpec((_BT, 352), lambda i: (i, 0)),
            pl.BlockSpec((_BT, M), lambda i: (i, 0)),
            _full((V, H)), _full((M - 1, VP, H)), _full((352, H)),
            _full((M, H)), _full((1, H)),
            _full((V, H)), _full((M - 1, VP, H)), _full((352, H)),
            _full((M, H)), _full((1, H)),
            _full((H, V)), _full((1, V)),
        ],
        out_specs=pl.BlockSpec((_BT, V), lambda i: (i, 0)),
        out_shape=jax.ShapeDtypeStruct((B, V), jnp.float32),
        compiler_params=pltpu.CompilerParams(
            dimension_semantics=("arbitrary",)),
    )(x, lp, sm, bt, ss, w1x, w1b, w1bt, w1ss, b1,
      wgx, wgb, wgbt, wgss, bg, w2, b2)


def kernel(x, memory, memory_timings, memory_surprise, last_prediction,
           W1, b1, Wg, bg, W2, b2):
    mt31 = memory_timings[:, : M - 1] + 1
    order = jnp.argsort(mt31, axis=1, stable=True)
    st = jnp.take_along_axis(mt31, order, axis=1)
    ssg = jnp.take_along_axis(memory_surprise[:, : M - 1], order, axis=1)
    idx = (jnp.arange(B, dtype=jnp.int32)[:, None] * (M - 1)
           + order.astype(jnp.int32)).reshape(_ROWS)
    stfull = jnp.concatenate(
        [jnp.zeros((B, 1), jnp.int32), st], axis=1)
    bits = ((stfull[:, :, None] >> jnp.arange(TIMING_DIM, dtype=jnp.int32))
            & 1).astype(jnp.float32).reshape(B, M * TIMING_DIM)
    norm_t = stfull.astype(jnp.float32) / (st[:, -1:].astype(jnp.float32) + 1.0)
    bt = jnp.concatenate([bits, norm_t], axis=1)
    ss = jnp.concatenate(
        [jnp.zeros((B, 1), jnp.float32), DECAY * ssg], axis=1)

    table = _pad_call(memory).reshape(_ROWS, VP)
    sorted31 = _sc_gather(idx, table).reshape(B, M - 1, VP)

    w1b = jnp.pad(W1[V:M * V].reshape(M - 1, V, H),
                  ((0, 0), (0, VP - V), (0, 0))).astype(jnp.bfloat16)
    wgb = jnp.pad(Wg[V:M * V].reshape(M - 1, V, H),
                  ((0, 0), (0, VP - V), (0, 0))).astype(jnp.bfloat16)
    w1x = W1[:V].astype(jnp.bfloat16)
    w1bt = W1[M * V:M * V + 352].astype(jnp.bfloat16)
    w1ss = W1[M * V + 352:]
    wgx = Wg[:V].astype(jnp.bfloat16)
    wgbt = Wg[M * V:M * V + 352].astype(jnp.bfloat16)
    wgss = Wg[M * V + 352:]
    w2 = W2.astype(jnp.bfloat16)

    return _mlp_call(x, last_prediction, sorted31, bt, ss,
                     w1x, w1b, w1bt, w1ss, b1.reshape(1, H),
                     wgx, wgb, wgbt, wgss, bg.reshape(1, H),
                     w2, b2.reshape(1, V))


# speedup vs baseline: 1.3614x; 1.3614x over previous
import jax
import jax.numpy as jnp
from jax import lax
from jax.experimental import pallas as pl
from jax.experimental.pallas import tpu as pltpu
from jax.experimental.pallas import tpu_sc as plsc

B = 1024
V = 1000
VP = 1024
M = 32
H = 256
TIMING_DIM = 10
DECAY = 0.99

_NC, _NS = 2, 16
_NW = _NC * _NS
_ROWS = B * M
_PER_W = _ROWS // _NW
_CHUNK = 32
_NCHUNK = _PER_W // _CHUNK


_BP = 64


def _pad_body(x_ref, mem_ref, out_ref):
    out_ref[:, 0, :V] = x_ref[...]
    out_ref[:, 1:, :V] = mem_ref[:, : M - 1, :]
    out_ref[:, :, V:] = jnp.zeros((_BP, M, VP - V), jnp.float32)


def _pad_call(x, memory):
    return pl.pallas_call(
        _pad_body,
        grid=(B // _BP,),
        in_specs=[
            pl.BlockSpec((_BP, V), lambda i: (i, 0)),
            pl.BlockSpec((_BP, M, V), lambda i: (i, 0, 0)),
        ],
        out_specs=pl.BlockSpec((_BP, M, VP), lambda i: (i, 0, 0)),
        out_shape=jax.ShapeDtypeStruct((B, M, VP), jnp.float32),
        compiler_params=pltpu.CompilerParams(
            dimension_semantics=("arbitrary",)),
    )(x, memory)


def _sc_gather_body(idx_hbm, table_hbm, out_hbm, idx_v, rows0, rows1, gsem,
                    wsem):
    wid = lax.axis_index("s") * _NC + lax.axis_index("c")
    base = wid * _PER_W
    pltpu.sync_copy(idx_hbm.at[pl.ds(base, _PER_W)], idx_v)
    rows = (rows0, rows1)
    writes = [None] * _NCHUNK
    gathers = [None] * _NCHUNK

    def start_gather(c):
        gathers[c] = pltpu.async_copy(
            table_hbm.at[idx_v.at[pl.ds(c * _CHUNK, _CHUNK)]],
            rows[c % 2], gsem)

    start_gather(0)
    for c in range(_NCHUNK):
        gathers[c].wait()
        if c + 1 < _NCHUNK:
            if c >= 1:
                writes[c - 1].wait()
            start_gather(c + 1)
        writes[c] = pltpu.async_copy(
            rows[c % 2], out_hbm.at[pl.ds(base + c * _CHUNK, _CHUNK)], wsem)
    writes[_NCHUNK - 2].wait()
    writes[_NCHUNK - 1].wait()


def _sc_gather(idx, table):
    return pl.kernel(
        _sc_gather_body,
        out_type=jax.ShapeDtypeStruct((_ROWS, VP), jnp.float32),
        mesh=plsc.VectorSubcoreMesh(core_axis_name="c", subcore_axis_name="s"),
        scratch_types=[
            pltpu.VMEM((_PER_W,), jnp.int32),
            pltpu.VMEM((_CHUNK, VP), jnp.float32),
            pltpu.VMEM((_CHUNK, VP), jnp.float32),
            pltpu.SemaphoreType.DMA,
            pltpu.SemaphoreType.DMA,
        ],
    )(idx, table)


_BT = 64
_NB = B // _BT
_NPH = 4
_MH = M // _NPH


def _mlp_body(x_ref, lp_ref, sm_ref, bt_ref, ss_ref,
              w1s_ref, w1bt_ref, w1ss_ref, b1_ref,
              wgs_ref, wgbt_ref, wgss_ref, bg_ref,
              w2_ref, b2_ref, out_ref, acc1_ref, acc2_ref):
    k = pl.program_id(0)
    i = pl.program_id(1)
    a1 = jnp.zeros((_BT, H), jnp.float32)
    a2 = jnp.zeros((_BT, H), jnp.float32)
    for j in range(_MH):
        smj = sm_ref[:, j, :V]
        a1 += jnp.dot(smj, w1s_ref[j], preferred_element_type=jnp.float32)
        a2 += jnp.dot(smj, wgs_ref[j], preferred_element_type=jnp.float32)

    @pl.when(k == 0)
    def _phase0():
        btb = bt_ref[...]
        b1t = a1 + jnp.dot(btb, w1bt_ref[...],
                           preferred_element_type=jnp.float32)
        b2t = a2 + jnp.dot(btb, wgbt_ref[...],
                           preferred_element_type=jnp.float32)
        surprise = -jnp.log(
            jnp.sum(x_ref[...] * lp_ref[...], axis=1, keepdims=True) + 1e-08)
        ss = ss_ref[...]
        b1t += jnp.dot(ss, w1ss_ref[...], preferred_element_type=jnp.float32)
        b2t += jnp.dot(ss, wgss_ref[...], preferred_element_type=jnp.float32)
        b1t += surprise * w1ss_ref[0:1, :]
        b2t += surprise * wgss_ref[0:1, :]
        acc1_ref[i] = b1t
        acc2_ref[i] = b2t

    @pl.when(jnp.logical_and(k > 0, k < _NPH - 1))
    def _mid():
        acc1_ref[i] = acc1_ref[i] + a1
        acc2_ref[i] = acc2_ref[i] + a2

    @pl.when(k == _NPH - 1)
    def _phase1():
        t1 = acc1_ref[i] + a1 + b1_ref[...]
        t2 = acc2_ref[i] + a2 + bg_ref[...]
        h = t1 * jax.nn.sigmoid(t2)
        out_ref[...] = (
            jnp.dot(h, w2_ref[...], preferred_element_type=jnp.float32)
            + b2_ref[...])


def _mlp_call(x, lp, sm, bt, ss, w1s, w1bt, w1ss, b1,
              wgs, wgbt, wgss, bg, w2, b2):
    def _c(shape):
        return pl.BlockSpec(shape, lambda k, i: (0,) * len(shape))

    return pl.pallas_call(
        _mlp_body,
        grid=(_NPH, _NB),
        in_specs=[
            pl.BlockSpec((_BT, V), lambda k, i: (i, 0)),
            pl.BlockSpec((_BT, V), lambda k, i: (i, 0)),
            pl.BlockSpec((_BT, _MH, VP), lambda k, i: (i, k, 0)),
            pl.BlockSpec((_BT, 352), lambda k, i: (i, 0)),
            pl.BlockSpec((_BT, M), lambda k, i: (i, 0)),
            pl.BlockSpec((_MH, V, H), lambda k, i: (k, 0, 0)),
            _c((352, H)), _c((M, H)), _c((1, H)),
            pl.BlockSpec((_MH, V, H), lambda k, i: (k, 0, 0)),
            _c((352, H)), _c((M, H)), _c((1, H)),
            _c((H, V)), _c((1, V)),
        ],
        out_specs=pl.BlockSpec((_BT, V), lambda k, i: (i, 0)),
        out_shape=jax.ShapeDtypeStruct((B, V), jnp.float32),
        scratch_shapes=[
            pltpu.VMEM((_NB, _BT, H), jnp.float32),
            pltpu.VMEM((_NB, _BT, H), jnp.float32),
        ],
        compiler_params=pltpu.CompilerParams(
            dimension_semantics=("arbitrary", "arbitrary")),
    )(x, lp, sm, bt, ss, w1s, w1bt, w1ss, b1,
      wgs, wgbt, wgss, bg, w2, b2)


def kernel(x, memory, memory_timings, memory_surprise, last_prediction,
           W1, b1, Wg, bg, W2, b2):
    mt31 = memory_timings[:, : M - 1] + 1
    order = jnp.argsort(mt31, axis=1, stable=True)
    st = jnp.take_along_axis(mt31, order, axis=1)
    ssg = jnp.take_along_axis(memory_surprise[:, : M - 1], order, axis=1)
    sidx = jnp.concatenate(
        [jnp.zeros((B, 1), jnp.int32), order.astype(jnp.int32) + 1], axis=1)
    idx = (jnp.arange(B, dtype=jnp.int32)[:, None] * M + sidx).reshape(_ROWS)
    stfull = jnp.concatenate(
        [jnp.zeros((B, 1), jnp.int32), st], axis=1)
    bits = ((stfull[:, :, None] >> jnp.arange(TIMING_DIM, dtype=jnp.int32))
            & 1).astype(jnp.float32).reshape(B, M * TIMING_DIM)
    norm_t = stfull.astype(jnp.float32) / (st[:, -1:].astype(jnp.float32) + 1.0)
    bt = jnp.concatenate([bits, norm_t], axis=1)
    ss = jnp.concatenate(
        [jnp.zeros((B, 1), jnp.float32), DECAY * ssg], axis=1)

    table = _pad_call(x, memory).reshape(_ROWS, VP)
    sorted_mem = _sc_gather(idx, table).reshape(B, M, VP)

    w1s = W1[:M * V].reshape(M, V, H)
    wgs = Wg[:M * V].reshape(M, V, H)
    w1bt = W1[M * V:M * V + 352]
    w1ss = W1[M * V + 352:]
    wgbt = Wg[M * V:M * V + 352]
    wgss = Wg[M * V + 352:]

    return _mlp_call(x, last_prediction, sorted_mem, bt, ss,
                     w1s, w1bt, w1ss, b1.reshape(1, H),
                     wgs, wgbt, wgss, bg.reshape(1, H),
                     W2, b2.reshape(1, V))
